# baseline (speedup 1.0000x reference)
import functools

import jax
import jax.numpy as jnp
from jax import lax
from jax.experimental import pallas as pl
from jax.experimental.pallas import tpu as pltpu
from jax.experimental.pallas import tpu_sc as plsc

NC = 2
NS = 16
NW = NC * NS
L = 16
ROWS = 128
COLS = 100000
SUB = 16
NB = ROWS // SUB
NQ = NW // NB
CW = 13 * 128
NCHUNK = 6
SCK0 = 60 - NQ * NCHUNK
TC_COLS = SCK0 * CW
TAIL0 = 60 * CW
TAILW = COLS - TAIL0

_mesh = plsc.VectorSubcoreMesh(core_axis_name="c", subcore_axis_name="s")


def _scan(buf, col0, ncols, accv, acci, iota):
    def body(i, carry):
        accv, acci, cur = carry
        nv, ni = [], []
        for s in range(SUB):
            v = buf[s, pl.ds(i * L, L)]
            pred = v > accv[s]
            nv.append(jnp.where(pred, v, accv[s]))
            ni.append(jnp.where(pred, cur, acci[s]))
        return tuple(nv), tuple(ni), cur + L

    accv, acci, _ = lax.fori_loop(
        0, ncols // L, body, (accv, acci, iota + col0), unroll=1
    )
    return accv, acci


@functools.partial(
    pl.kernel,
    out_type=(
        jax.ShapeDtypeStruct((NW, L), jnp.float32),
        jax.ShapeDtypeStruct((NW, L), jnp.int32),
    ),
    mesh=_mesh,
    compiler_params=pltpu.CompilerParams(use_tc_tiling_on_sc=True),
    scratch_types=[
        pltpu.VMEM((SUB, CW), jnp.float32),
        pltpu.VMEM((SUB, CW), jnp.float32),
        pltpu.VMEM((SUB, TAILW), jnp.float32),
        pltpu.VMEM((L,), jnp.float32),
        pltpu.VMEM((L,), jnp.int32),
        pltpu.SemaphoreType.DMA,
        pltpu.SemaphoreType.DMA,
        pltpu.SemaphoreType.DMA,
    ],
)
def _argmax_sc(x_hbm, outv_hbm, outi_hbm, buf0, buf1, tailbuf, resv, resi,
               sem0, sem1, semt):
    wid = lax.axis_index("s") * NC + lax.axis_index("c")
    b = wid // NQ
    q = wid % NQ
    iota = lax.iota(jnp.int32, L)
    rows = pl.ds(b * SUB, SUB)

    def chunk_col0(i):
        return (SCK0 + NQ * i + q) * CW

    def chunk_src(i):
        return x_hbm.at[rows, pl.ds(chunk_col0(i), CW)]

    bufs = (buf0, buf1)
    sems = (sem0, sem1)

    pltpu.async_copy(chunk_src(0), buf0, sem0)
    pltpu.async_copy(x_hbm.at[rows, pl.ds(TAIL0, TAILW)], tailbuf, semt)

    accv = tuple(jnp.full((L,), -jnp.inf, jnp.float32) for _ in range(SUB))
    acci = tuple(jnp.zeros((L,), jnp.int32) for _ in range(SUB))

    for i in range(NCHUNK):
        s = i & 1
        if i + 1 < NCHUNK:
            pltpu.async_copy(chunk_src(i + 1), bufs[1 - s], sems[1 - s])
        pltpu.make_async_copy(chunk_src(i), bufs[s], sems[s]).wait()
        accv, acci = _scan(bufs[s], chunk_col0(i), CW, accv, acci, iota)

    pltpu.make_async_copy(
        x_hbm.at[rows, pl.ds(TAIL0, TAILW)], tailbuf, semt
    ).wait()
    accv, acci = _scan(tailbuf, TAIL0, TAILW, accv, acci, iota)

    resv_vec = jnp.zeros((L,), jnp.float32)
    resi_vec = jnp.zeros((L,), jnp.int32)
    for s in range(SUB):
        bm, bi = accv[s], acci[s]
        best_v = bm[0]
        best_i = bi[0]
        for k in range(1, L):
            pv = bm[k]
            pi = bi[k]
            pred = (pv > best_v) | ((pv == best_v) & (pi < best_i))
            best_v = jnp.where(pred, pv, best_v)
            best_i = jnp.where(pred, pi, best_i)
        resv_vec = jnp.where(iota == s, best_v, resv_vec)
        resi_vec = jnp.where(iota == s, best_i, resi_vec)

    resv[...] = resv_vec
    resi[...] = resi_vec
    pltpu.sync_copy(resv, outv_hbm.at[wid])
    pltpu.sync_copy(resi, outi_hbm.at[wid])


BLK = CW
BIG = 2**31 - 1


def _tc_body(x_ref, outv_ref, outi_ref, mv_ref, mi_ref):
    k = pl.program_id(0)
    blk = x_ref[...]
    m = jnp.max(blk, axis=1, keepdims=True)
    cols = lax.broadcasted_iota(jnp.int32, blk.shape, 1) + k * BLK
    idx = jnp.min(jnp.where(blk == m, cols, jnp.int32(BIG)), axis=1, keepdims=True)

    @pl.when(k == 0)
    def _():
        mv_ref[...] = m
        mi_ref[...] = idx

    @pl.when(k > 0)
    def _():
        better = m > mv_ref[...]
        mv_ref[...] = jnp.where(better, m, mv_ref[...])
        mi_ref[...] = jnp.where(better, idx, mi_ref[...])

    @pl.when(k == pl.num_programs(0) - 1)
    def _():
        outv_ref[...] = mv_ref[...]
        outi_ref[...] = mi_ref[...]


_argmax_tc = pl.pallas_call(
    _tc_body,
    grid=(SCK0,),
    in_specs=[pl.BlockSpec((ROWS, BLK), lambda k: (0, k))],
    out_specs=(
        pl.BlockSpec((ROWS, 1), lambda k: (0, 0)),
        pl.BlockSpec((ROWS, 1), lambda k: (0, 0)),
    ),
    out_shape=(
        jax.ShapeDtypeStruct((ROWS, 1), jnp.float32),
        jax.ShapeDtypeStruct((ROWS, 1), jnp.int32),
    ),
    scratch_shapes=[
        pltpu.VMEM((ROWS, 1), jnp.float32),
        pltpu.VMEM((ROWS, 1), jnp.int32),
    ],
)


def kernel(m_logits):
    outv, outi = _argmax_sc(m_logits)
    tcv, tci = _argmax_tc(m_logits)
    v = outv.reshape(NB, NQ, L)
    i = outi.reshape(NB, NQ, L)
    bv, bi = v[:, 0], i[:, 0]
    for qq in range(1, NQ):
        pred = (v[:, qq] > bv) | ((v[:, qq] == bv) & (i[:, qq] < bi))
        bv = jnp.where(pred, v[:, qq], bv)
        bi = jnp.where(pred, i[:, qq], bi)
    scv = bv.reshape(ROWS, 1)
    sci = bi.reshape(ROWS, 1)
    pred = scv > tcv[..., 0].reshape(ROWS, 1)
    idx = jnp.where(pred, sci, tci[..., 0].reshape(ROWS, 1))
    return idx


# speedup vs baseline: 2.1355x; 2.1355x over previous
import functools

import jax
import jax.numpy as jnp
from jax import lax
from jax.experimental import pallas as pl
from jax.experimental.pallas import tpu as pltpu
from jax.experimental.pallas import tpu_sc as plsc

NC = 2
NS = 16
NW = NC * NS
L = 16
G = 8
ROWS = 128
COLS = 100000
R = 400
NCH = COLS // R
CPW = 8

_mesh = plsc.VectorSubcoreMesh(core_axis_name="c", subcore_axis_name="s")


@functools.partial(
    pl.kernel,
    out_type=(
        jax.ShapeDtypeStruct((NW, G * L), jnp.float32),
        jax.ShapeDtypeStruct((NW, G * L), jnp.int32),
    ),
    mesh=_mesh,
    compiler_params=pltpu.CompilerParams(use_tc_tiling_on_sc=True),
    scratch_types=[
        pltpu.VMEM((R, ROWS), jnp.float32),
        pltpu.VMEM((R, ROWS), jnp.float32),
        pltpu.VMEM((G * L,), jnp.float32),
        pltpu.VMEM((G * L,), jnp.int32),
        pltpu.SemaphoreType.DMA,
        pltpu.SemaphoreType.DMA,
    ],
)
def _argmax_sc(xt_hbm, outv_hbm, outi_hbm, buf0, buf1, resv, resi, sem0, sem1):
    wid = lax.axis_index("s") * NC + lax.axis_index("c")

    def chunk_r0(i):
        return jnp.minimum(wid + NW * i, NCH - 1) * R

    def chunk_src(i):
        return xt_hbm.at[pl.ds(chunk_r0(i), R), :]

    bufs = (buf0, buf1)
    sems = (sem0, sem1)

    pltpu.async_copy(chunk_src(0), buf0, sem0)

    accv = tuple(jnp.full((L,), -jnp.inf, jnp.float32) for _ in range(G))
    acci = tuple(jnp.zeros((L,), jnp.int32) for _ in range(G))

    for i in range(CPW):
        s = i & 1
        if i + 1 < CPW:
            pltpu.async_copy(chunk_src(i + 1), bufs[1 - s], sems[1 - s])
        pltpu.make_async_copy(chunk_src(i), bufs[s], sems[s]).wait()

        def body(k, carry, buf=bufs[s]):
            accv, acci, cur = carry
            nv, ni = [], []
            for g in range(G):
                v = buf[k, pl.ds(g * L, L)]
                pred = v > accv[g]
                nv.append(jnp.where(pred, v, accv[g]))
                ni.append(jnp.where(pred, cur, acci[g]))
            return tuple(nv), tuple(ni), cur + 1

        cur0 = jnp.full((L,), chunk_r0(i), jnp.int32)
        accv, acci, _ = lax.fori_loop(
            0, R, body, (accv, acci, cur0), unroll=2
        )

    for g in range(G):
        resv[pl.ds(g * L, L)] = accv[g]
        resi[pl.ds(g * L, L)] = acci[g]
    pltpu.sync_copy(resv, outv_hbm.at[wid])
    pltpu.sync_copy(resi, outi_hbm.at[wid])


def kernel(m_logits):
    outv, outi = _argmax_sc(m_logits.T)
    bv, bi = outv[0], outi[0]
    for w in range(1, NW):
        pred = (outv[w] > bv) | ((outv[w] == bv) & (outi[w] < bi))
        bv = jnp.where(pred, outv[w], bv)
        bi = jnp.where(pred, outi[w], bi)
    return bi.reshape(ROWS, 1)
